# baseline (speedup 1.0000x reference)
import functools

import jax
import jax.numpy as jnp
from jax import lax
from jax.experimental import pallas as pl
from jax.experimental.pallas import tpu as pltpu
from jax.experimental.pallas import tpu_sc as plsc

NC = 2
NS = 16
L = 16
NW = NC * NS

D = 16
CHUNK = 128
L2_ALPHA = 0.01


@functools.cache
def _build(batch, num_users, num_items):
    p1 = batch // NW
    p1k = p1 // CHUNK
    p2 = batch // NS
    p2k = p2 // CHUNK
    mesh = plsc.VectorSubcoreMesh(core_axis_name="c", subcore_axis_name="s")

    @functools.partial(
        pl.kernel,
        out_type=[
            jax.ShapeDtypeStruct((NW, L), jnp.float32),
            jax.ShapeDtypeStruct((num_users,), jnp.int32),
            jax.ShapeDtypeStruct((num_items,), jnp.int32),
        ],
        mesh=mesh,
        compiler_params=pltpu.CompilerParams(needs_layout_passes=False,
                                             use_tc_tiling_on_sc=False),
        scratch_types=[
            pltpu.VMEM((p1k, CHUNK), jnp.int32),
            pltpu.VMEM((p1k, CHUNK), jnp.int32),
            pltpu.VMEM((p1, D), jnp.float32),
            pltpu.VMEM((p1, D), jnp.float32),
            pltpu.VMEM((p1,), jnp.float32),
            pltpu.VMEM((p2k, CHUNK), jnp.int32),
            pltpu.VMEM((p2k, CHUNK), jnp.int32),
            pltpu.VMEM((CHUNK,), jnp.int32),
            pltpu.VMEM((CHUNK, D), jnp.float32),
            pltpu.VMEM((L,), jnp.float32),
            pltpu.SemaphoreType.DMA,
        ],
    )
    def k(users, items, scores, bpos, ut, it, out, slot_u, slot_i,
          uidx, iidx, urows, irows, svals, didx, dpos, wbuf, drows, pbuf,
          sem):
        c = lax.axis_index("c")
        s = lax.axis_index("s")
        wid = s * NC + c

        base = wid * p1
        for kk in range(p1k):
            pltpu.sync_copy(users.at[pl.ds(base + kk * CHUNK, CHUNK)],
                            uidx.at[kk])
            pltpu.sync_copy(items.at[pl.ds(base + kk * CHUNK, CHUNK)],
                            iidx.at[kk])
        pltpu.sync_copy(scores.at[pl.ds(base, p1)], svals)
        for kk in range(p1k):
            pltpu.async_copy(ut.at[uidx.at[kk]],
                             urows.at[pl.ds(kk * CHUNK, CHUNK), :],
                             sem).wait()
            pltpu.async_copy(it.at[iidx.at[kk]],
                             irows.at[pl.ds(kk * CHUNK, CHUNK), :],
                             sem).wait()

        def body1(jb, sse):
            sv = svals[pl.ds(jb * L, L)]
            for q in range(L):
                u = urows[jb * L + q]
                v = irows[jb * L + q]
                pred = jnp.sum(u * v)
                dd = pred - sv[q]
                sse = sse + dd * dd
            return sse

        sse = lax.fori_loop(0, p1 // L, body1, jnp.float32(0.0))

        base2 = s * p2

        def scatter_phase(idx_hbm, slots):
            for kk in range(p2k):
                pltpu.sync_copy(idx_hbm.at[pl.ds(base2 + kk * CHUNK, CHUNK)],
                                didx.at[kk])
                pltpu.sync_copy(bpos.at[pl.ds(base2 + kk * CHUNK, CHUNK)],
                                dpos.at[kk])
                pltpu.async_copy(dpos.at[kk], slots.at[didx.at[kk]],
                                 sem).wait()

        @pl.when(c == 0)
        def _():
            scatter_phase(users, slot_u)

        @pl.when(c == 1)
        def _():
            scatter_phase(items, slot_i)

        plsc.subcore_barrier()

        def gather_phase(table, slots):
            nacc = jnp.zeros((L,), jnp.float32)
            for kk in range(p2k):
                pltpu.async_copy(slots.at[didx.at[kk]], wbuf, sem).wait()
                pltpu.async_copy(table.at[didx.at[kk]], drows, sem).wait()

                def body2(gb, acc):
                    wv = wbuf[pl.ds(gb * L, L)]
                    pv = dpos[kk, pl.ds(gb * L, L)]
                    mvec = jnp.where(wv == pv, jnp.float32(1.0),
                                     jnp.float32(0.0))
                    for q in range(L):
                        r = drows[gb * L + q]
                        acc = acc + (r * r) * mvec[q]
                    return acc

                nacc = lax.fori_loop(0, CHUNK // L, body2, nacc)
            l2 = jnp.sum(nacc)
            partial = sse + jnp.float32(0.5 * L2_ALPHA) * l2
            lane = lax.iota(jnp.int32, L)
            pbuf[...] = jnp.where(lane == 0, partial, jnp.float32(0.0))
            pltpu.sync_copy(pbuf, out.at[wid])

        @pl.when(c == 0)
        def _():
            gather_phase(ut, slot_u)

        @pl.when(c == 1)
        def _():
            gather_phase(it, slot_i)

    return k


def kernel(users, items, scores, user_table, item_table, user_bias,
           item_bias):
    del user_bias, item_bias
    batch = users.shape[0]
    bpos = jnp.arange(batch, dtype=jnp.int32)
    k = _build(batch, user_table.shape[0], item_table.shape[0])
    out, _, _ = k(users.astype(jnp.int32), items.astype(jnp.int32),
                  scores, bpos, user_table, item_table)
    return jnp.sum(out)
